# Initial kernel scaffold; baseline (speedup 1.0000x reference)
import jax
import jax.numpy as jnp
from jax.experimental import pallas as pl


def kernel(input, Adj, w_diag1, w_diag2, W1, b1, W2, b2):
    raise NotImplementedError("write your pallas kernel here")


# speedup vs baseline: 13.6702x; 13.6702x over previous
import functools

import jax
import jax.numpy as jnp
from jax.experimental import pallas as pl
from jax.experimental.pallas import tpu as pltpu

N = 10000
F = 256
H = 128
C = 40
TOPK = 50

_RB = 400
_RBD = 80

def _f2k(x):
    b = jax.lax.bitcast_convert_type(x, jnp.int32)
    imin = jnp.full_like(b, -2147483648)
    return jnp.where(b >= 0, b, imin - b)


def _k2f(k):
    imin = jnp.full_like(k, -2147483648)
    b = jnp.where(k >= 0, k, imin - k)
    return jax.lax.bitcast_convert_type(b, jnp.float32)


def _layer_kernel(a_ref, dr_ref, dc_ref, m_ref, o_ref):
    scaled = (dr_ref[...] * a_ref[...]) * dc_ref[...]
    o_ref[...] = jnp.dot(scaled, m_ref[...], preferred_element_type=jnp.float32)


def _layer(adj, dinv_col, dinv_row, m):
    return pl.pallas_call(
        _layer_kernel,
        grid=(N // _RB,),
        in_specs=[
            pl.BlockSpec((_RB, N), lambda i: (i, 0)),
            pl.BlockSpec((_RB, 1), lambda i: (i, 0)),
            pl.BlockSpec((1, N), lambda i: (0, 0)),
            pl.BlockSpec((N, F), lambda i: (0, 0)),
        ],
        out_specs=pl.BlockSpec((_RB, F), lambda i: (i, 0)),
        out_shape=jax.ShapeDtypeStruct((N, F), jnp.float32),
    )(adj, dinv_col, dinv_row, m)


def _simsel_kernel(e_ref, et_ref, t_ref, c_ref, sim_ref):
    e = e_ref[...]
    et = et_ref[...]
    half = F // 2
    sim = (jnp.dot(e[:, :half], et[:half, :], preferred_element_type=jnp.float32)
           + jnp.dot(e[:, half:], et[half:, :], preferred_element_type=jnp.float32))
    sim_ref[...] = sim

    rmin = jnp.min(sim, axis=1, keepdims=True)
    rmax = jnp.max(sim, axis=1, keepdims=True)
    lo0 = _f2k(rmin)
    hi0 = _f2k(rmax) + 1

    def vcond(carry):
        lo, hi = carry
        return jnp.any(hi - lo > 1)

    def vbody(carry):
        lo, hi = carry
        mid = jax.lax.shift_right_arithmetic(lo + hi, 1)
        cnt = jnp.sum((sim_ref[...] >= _k2f(mid)).astype(jnp.float32),
                      axis=1, keepdims=True)
        feas = cnt >= TOPK
        return (jnp.where(feas, mid, lo), jnp.where(feas, hi, mid))

    lo, _ = jax.lax.while_loop(vcond, vbody, (lo0, hi0))
    t = _k2f(lo)

    simv = sim_ref[...]
    n_gt = jnp.sum((simv > t).astype(jnp.float32), axis=1, keepdims=True)
    n_tie = jnp.float32(TOPK) - n_gt

    colf = jax.lax.broadcasted_iota(jnp.int32, simv.shape, 1).astype(jnp.float32)
    val = jnp.where(simv == t, colf, jnp.float32(3e9))

    def ccond(carry):
        lo2, hi2 = carry
        return jnp.any(hi2 - lo2 > 1)

    def cbody(carry):
        lo2, hi2 = carry
        mid = jax.lax.shift_right_arithmetic(lo2 + hi2, 1)
        cnt = jnp.sum((val <= mid.astype(jnp.float32)).astype(jnp.float32),
                      axis=1, keepdims=True)
        feas = cnt >= n_tie
        return (jnp.where(feas, lo2, mid), jnp.where(feas, mid, hi2))

    lo2 = jnp.full(t.shape, -1, jnp.int32)
    hi2 = jnp.full(t.shape, N - 1, jnp.int32)
    _, hi2 = jax.lax.while_loop(ccond, cbody, (lo2, hi2))

    t_ref[...] = t
    c_ref[...] = hi2.astype(jnp.float32)


def _simsel(emb, embT):
    return pl.pallas_call(
        _simsel_kernel,
        grid=(N // _RB,),
        in_specs=[
            pl.BlockSpec((_RB, F), lambda i: (i, 0)),
            pl.BlockSpec((F, N), lambda i: (0, 0)),
        ],
        out_specs=[
            pl.BlockSpec((_RB, 1), lambda i: (i, 0)),
            pl.BlockSpec((_RB, 1), lambda i: (i, 0)),
        ],
        out_shape=[
            jax.ShapeDtypeStruct((N, 1), jnp.float32),
            jax.ShapeDtypeStruct((N, 1), jnp.float32),
        ],
        scratch_shapes=[pltpu.VMEM((_RB, N), jnp.float32)],
    )(emb, embT)


def _graph_kernel(a_ref, e_ref, et_ref, tr_ref, cr_ref, tc_ref, cc_ref,
                  an_ref, af_ref, d2_ref):
    i = pl.program_id(0)
    e = e_ref[...]
    et = et_ref[...]
    half = F // 2
    sim = (jnp.dot(e[:, :half], et[:half, :], preferred_element_type=jnp.float32)
           + jnp.dot(e[:, half:], et[half:, :], preferred_element_type=jnp.float32))

    colf = jax.lax.broadcasted_iota(jnp.int32, sim.shape, 1).astype(jnp.float32)
    rowf = (jax.lax.broadcasted_iota(jnp.int32, sim.shape, 0).astype(jnp.float32)
            + jnp.float32(_RBD) * i.astype(jnp.float32))

    tr = tr_ref[...]
    cr = cr_ref[...]
    tc = tc_ref[...]
    cc = cc_ref[...]

    keep_r = jnp.logical_or(sim > tr, jnp.logical_and(sim == tr, colf <= cr))
    keep_c = jnp.logical_or(sim > tc, jnp.logical_and(sim == tc, rowf <= cc))
    fac = 0.5 * (keep_r.astype(jnp.float32) + keep_c.astype(jnp.float32))
    anew = sim * fac
    af = anew + a_ref[...]
    an_ref[...] = anew
    af_ref[...] = af
    d2_ref[...] = jnp.sum(af, axis=1, keepdims=True)


def _graph(adj, emb, embT, t_col, c_col, t_row, c_row):
    return pl.pallas_call(
        _graph_kernel,
        grid=(N // _RBD,),
        in_specs=[
            pl.BlockSpec((_RBD, N), lambda i: (i, 0)),
            pl.BlockSpec((_RBD, F), lambda i: (i, 0)),
            pl.BlockSpec((F, N), lambda i: (0, 0)),
            pl.BlockSpec((_RBD, 1), lambda i: (i, 0)),
            pl.BlockSpec((_RBD, 1), lambda i: (i, 0)),
            pl.BlockSpec((1, N), lambda i: (0, 0)),
            pl.BlockSpec((1, N), lambda i: (0, 0)),
        ],
        out_specs=[
            pl.BlockSpec((_RBD, N), lambda i: (i, 0)),
            pl.BlockSpec((_RBD, N), lambda i: (i, 0)),
            pl.BlockSpec((_RBD, 1), lambda i: (i, 0)),
        ],
        out_shape=[
            jax.ShapeDtypeStruct((N, N), jnp.float32),
            jax.ShapeDtypeStruct((N, N), jnp.float32),
            jax.ShapeDtypeStruct((N, 1), jnp.float32),
        ],
    )(adj, emb, embT, t_col, c_col, t_row, c_row)


def _mm_kernel(x_ref, w_ref, o_ref):
    o_ref[...] = jnp.dot(x_ref[...], w_ref[...],
                         preferred_element_type=jnp.float32)


def _mm(x, w):
    kin, nout = w.shape
    return pl.pallas_call(
        _mm_kernel,
        grid=(N // _RB,),
        in_specs=[
            pl.BlockSpec((_RB, kin), lambda i: (i, 0)),
            pl.BlockSpec((kin, nout), lambda i: (0, 0)),
        ],
        out_specs=pl.BlockSpec((_RB, nout), lambda i: (i, 0)),
        out_shape=jax.ShapeDtypeStruct((N, nout), jnp.float32),
    )(x, w)


def _enc_kernel(relu, af_ref, z_ref, d_ref, b_ref, o_ref):
    acc = jnp.dot(af_ref[...], z_ref[...], preferred_element_type=jnp.float32)
    r = d_ref[...] * acc + b_ref[...]
    if relu:
        r = jnp.maximum(r, 0.0)
    o_ref[...] = r


def _enc(af, z, dinv2, b, relu):
    nout = z.shape[1]
    return pl.pallas_call(
        functools.partial(_enc_kernel, relu),
        grid=(N // _RB,),
        in_specs=[
            pl.BlockSpec((_RB, N), lambda i: (i, 0)),
            pl.BlockSpec((N, nout), lambda i: (0, 0)),
            pl.BlockSpec((_RB, 1), lambda i: (i, 0)),
            pl.BlockSpec((1, nout), lambda i: (0, 0)),
        ],
        out_specs=pl.BlockSpec((_RB, nout), lambda i: (i, 0)),
        out_shape=jax.ShapeDtypeStruct((N, nout), jnp.float32),
    )(af, z, dinv2, b)


def kernel(input, Adj, w_diag1, w_diag2, W1, b1, W2, b2):
    deg = jnp.sum(Adj, axis=1)
    dinv = jnp.where(deg > 0, deg ** -0.5, 0.0)
    dcol = dinv.reshape(N, 1)
    drow = dinv.reshape(1, N)

    h = _layer(Adj, dcol, drow, input * w_diag1)
    h = _layer(Adj, dcol, drow, h * w_diag2)

    nrm = jnp.linalg.norm(h, axis=1, keepdims=True)
    emb = h / jnp.maximum(nrm, 1e-12)
    embT = emb.T

    t, cstar = _simsel(emb, embT)

    adj_new, adj_final, deg2 = _graph(
        Adj, emb, embT, t, cstar, t.reshape(1, N), cstar.reshape(1, N))

    dinv2 = jnp.where(deg2 > 0, deg2 ** -0.5, 0.0)

    z1 = _mm(input, W1) * dinv2
    h1 = _enc(adj_final, z1, dinv2, b1.reshape(1, H), True)
    z2 = _mm(h1, W2) * dinv2
    out = _enc(adj_final, z2, dinv2, b2.reshape(1, C), False)

    return (out, adj_new, adj_final)


# speedup vs baseline: 15.0540x; 1.1012x over previous
import functools

import jax
import jax.numpy as jnp
from jax.experimental import pallas as pl
from jax.experimental.pallas import tpu as pltpu

N = 10000
F = 256
H = 128
C = 40
TOPK = 50

_RB = 400
_RBD = 80

def _f2k(x):
    b = jax.lax.bitcast_convert_type(x, jnp.int32)
    imin = jnp.full_like(b, -2147483648)
    return jnp.where(b >= 0, b, imin - b)


def _k2f(k):
    imin = jnp.full_like(k, -2147483648)
    b = jnp.where(k >= 0, k, imin - k)
    return jax.lax.bitcast_convert_type(b, jnp.float32)


def _layer_kernel(a_ref, dr_ref, dc_ref, m_ref, o_ref):
    scaled = (dr_ref[...] * a_ref[...]) * dc_ref[...]
    o_ref[...] = jnp.dot(scaled, m_ref[...], preferred_element_type=jnp.float32)


def _layer(adj, dinv_col, dinv_row, m):
    return pl.pallas_call(
        _layer_kernel,
        grid=(N // _RB,),
        in_specs=[
            pl.BlockSpec((_RB, N), lambda i: (i, 0)),
            pl.BlockSpec((_RB, 1), lambda i: (i, 0)),
            pl.BlockSpec((1, N), lambda i: (0, 0)),
            pl.BlockSpec((N, F), lambda i: (0, 0)),
        ],
        out_specs=pl.BlockSpec((_RB, F), lambda i: (i, 0)),
        out_shape=jax.ShapeDtypeStruct((N, F), jnp.float32),
    )(adj, dinv_col, dinv_row, m)


def _simsel_kernel(e_ref, et_ref, tri_ref, t_ref, c_ref, sim_ref):
    e = e_ref[...]
    et = et_ref[...]
    half = F // 2
    sim = (jnp.dot(e[:, :half], et[:half, :], preferred_element_type=jnp.float32)
           + jnp.dot(e[:, half:], et[half:, :], preferred_element_type=jnp.float32))
    sim_ref[...] = sim

    rmin = jnp.min(sim, axis=1, keepdims=True)
    rmax = jnp.max(sim, axis=1, keepdims=True)
    lo0 = _f2k(rmin)
    hi0 = _f2k(rmax) + 1

    def vcond(carry):
        lo, hi = carry
        return jnp.any(hi - lo > 1)

    def vbody(carry):
        lo, hi = carry
        mid = jax.lax.shift_right_arithmetic(lo + hi, 1)
        cnt = jnp.sum((sim_ref[...] >= _k2f(mid)).astype(jnp.float32),
                      axis=1, keepdims=True)
        feas = cnt >= TOPK
        return (jnp.where(feas, mid, lo), jnp.where(feas, hi, mid))

    lo, _ = jax.lax.while_loop(vcond, vbody, (lo0, hi0))
    t = _k2f(lo)

    simv = sim_ref[...]
    n_gt = jnp.sum((simv > t).astype(jnp.float32), axis=1, keepdims=True)
    n_tie = jnp.float32(TOPK) - n_gt

    colf = jax.lax.broadcasted_iota(jnp.int32, simv.shape, 1).astype(jnp.float32)
    eqf = (simv == t).astype(jnp.float32)
    val = jnp.where(simv == t, colf, jnp.float32(3e9))

    pref = jnp.dot(eqf, tri_ref[...], preferred_element_type=jnp.float32)
    kstar = jnp.sum((pref < n_tie).astype(jnp.int32), axis=1, keepdims=True)
    lo2 = kstar * 79 - 1
    hi2 = jnp.minimum(kstar * 79 + 78, N - 1)

    def ccond(carry):
        lo2, hi2 = carry
        return jnp.any(hi2 - lo2 > 1)

    def cbody(carry):
        lo2, hi2 = carry
        mid = jax.lax.shift_right_arithmetic(lo2 + hi2, 1)
        cnt = jnp.sum((val <= mid.astype(jnp.float32)).astype(jnp.float32),
                      axis=1, keepdims=True)
        feas = cnt >= n_tie
        return (jnp.where(feas, lo2, mid), jnp.where(feas, mid, hi2))

    _, hi2 = jax.lax.while_loop(ccond, cbody, (lo2, hi2))

    t_ref[...] = t
    c_ref[...] = hi2.astype(jnp.float32)


def _simsel(emb, embT, tri):
    return pl.pallas_call(
        _simsel_kernel,
        grid=(N // _RB,),
        in_specs=[
            pl.BlockSpec((_RB, F), lambda i: (i, 0)),
            pl.BlockSpec((F, N), lambda i: (0, 0)),
            pl.BlockSpec((N, 128), lambda i: (0, 0)),
        ],
        out_specs=[
            pl.BlockSpec((_RB, 1), lambda i: (i, 0)),
            pl.BlockSpec((_RB, 1), lambda i: (i, 0)),
        ],
        out_shape=[
            jax.ShapeDtypeStruct((N, 1), jnp.float32),
            jax.ShapeDtypeStruct((N, 1), jnp.float32),
        ],
        scratch_shapes=[pltpu.VMEM((_RB, N), jnp.float32)],
    )(emb, embT, tri)


def _graph_kernel(a_ref, e_ref, et_ref, tr_ref, cr_ref, tc_ref, cc_ref,
                  an_ref, af_ref, d2_ref):
    i = pl.program_id(0)
    e = e_ref[...]
    et = et_ref[...]
    half = F // 2
    sim = (jnp.dot(e[:, :half], et[:half, :], preferred_element_type=jnp.float32)
           + jnp.dot(e[:, half:], et[half:, :], preferred_element_type=jnp.float32))

    colf = jax.lax.broadcasted_iota(jnp.int32, sim.shape, 1).astype(jnp.float32)
    rowf = (jax.lax.broadcasted_iota(jnp.int32, sim.shape, 0).astype(jnp.float32)
            + jnp.float32(_RBD) * i.astype(jnp.float32))

    tr = tr_ref[...]
    cr = cr_ref[...]
    tc = tc_ref[...]
    cc = cc_ref[...]

    keep_r = jnp.logical_or(sim > tr, jnp.logical_and(sim == tr, colf <= cr))
    keep_c = jnp.logical_or(sim > tc, jnp.logical_and(sim == tc, rowf <= cc))
    fac = 0.5 * (keep_r.astype(jnp.float32) + keep_c.astype(jnp.float32))
    anew = sim * fac
    af = anew + a_ref[...]
    an_ref[...] = anew
    af_ref[...] = af
    d2_ref[...] = jnp.sum(af, axis=1, keepdims=True)


def _graph(adj, emb, embT, t_col, c_col, t_row, c_row):
    return pl.pallas_call(
        _graph_kernel,
        grid=(N // _RBD,),
        in_specs=[
            pl.BlockSpec((_RBD, N), lambda i: (i, 0)),
            pl.BlockSpec((_RBD, F), lambda i: (i, 0)),
            pl.BlockSpec((F, N), lambda i: (0, 0)),
            pl.BlockSpec((_RBD, 1), lambda i: (i, 0)),
            pl.BlockSpec((_RBD, 1), lambda i: (i, 0)),
            pl.BlockSpec((1, N), lambda i: (0, 0)),
            pl.BlockSpec((1, N), lambda i: (0, 0)),
        ],
        out_specs=[
            pl.BlockSpec((_RBD, N), lambda i: (i, 0)),
            pl.BlockSpec((_RBD, N), lambda i: (i, 0)),
            pl.BlockSpec((_RBD, 1), lambda i: (i, 0)),
        ],
        out_shape=[
            jax.ShapeDtypeStruct((N, N), jnp.float32),
            jax.ShapeDtypeStruct((N, N), jnp.float32),
            jax.ShapeDtypeStruct((N, 1), jnp.float32),
        ],
    )(adj, emb, embT, t_col, c_col, t_row, c_row)


def _mm_kernel(x_ref, w_ref, o_ref):
    o_ref[...] = jnp.dot(x_ref[...], w_ref[...],
                         preferred_element_type=jnp.float32)


def _mm(x, w):
    kin, nout = w.shape
    return pl.pallas_call(
        _mm_kernel,
        grid=(N // _RB,),
        in_specs=[
            pl.BlockSpec((_RB, kin), lambda i: (i, 0)),
            pl.BlockSpec((kin, nout), lambda i: (0, 0)),
        ],
        out_specs=pl.BlockSpec((_RB, nout), lambda i: (i, 0)),
        out_shape=jax.ShapeDtypeStruct((N, nout), jnp.float32),
    )(x, w)


def _enc_kernel(relu, af_ref, z_ref, d_ref, b_ref, o_ref):
    acc = jnp.dot(af_ref[...], z_ref[...], preferred_element_type=jnp.float32)
    r = d_ref[...] * acc + b_ref[...]
    if relu:
        r = jnp.maximum(r, 0.0)
    o_ref[...] = r


def _enc(af, z, dinv2, b, relu):
    nout = z.shape[1]
    return pl.pallas_call(
        functools.partial(_enc_kernel, relu),
        grid=(N // _RB,),
        in_specs=[
            pl.BlockSpec((_RB, N), lambda i: (i, 0)),
            pl.BlockSpec((N, nout), lambda i: (0, 0)),
            pl.BlockSpec((_RB, 1), lambda i: (i, 0)),
            pl.BlockSpec((1, nout), lambda i: (0, 0)),
        ],
        out_specs=pl.BlockSpec((_RB, nout), lambda i: (i, 0)),
        out_shape=jax.ShapeDtypeStruct((N, nout), jnp.float32),
    )(af, z, dinv2, b)


def kernel(input, Adj, w_diag1, w_diag2, W1, b1, W2, b2):
    deg = jnp.sum(Adj, axis=1)
    dinv = jnp.where(deg > 0, deg ** -0.5, 0.0)
    dcol = dinv.reshape(N, 1)
    drow = dinv.reshape(1, N)

    h = _layer(Adj, dcol, drow, input * w_diag1)
    h = _layer(Adj, dcol, drow, h * w_diag2)

    nrm = jnp.linalg.norm(h, axis=1, keepdims=True)
    emb = h / jnp.maximum(nrm, 1e-12)
    embT = emb.T

    tri = (jnp.arange(N, dtype=jnp.int32)[:, None]
           <= (79 * jnp.arange(128, dtype=jnp.int32) + 78)[None, :]
           ).astype(jnp.float32)
    t, cstar = _simsel(emb, embT, tri)

    adj_new, adj_final, deg2 = _graph(
        Adj, emb, embT, t, cstar, t.reshape(1, N), cstar.reshape(1, N))

    dinv2 = jnp.where(deg2 > 0, deg2 ** -0.5, 0.0)

    z1 = _mm(input, W1) * dinv2
    h1 = _enc(adj_final, z1, dinv2, b1.reshape(1, H), True)
    z2 = _mm(h1, W2) * dinv2
    out = _enc(adj_final, z2, dinv2, b2.reshape(1, C), False)

    return (out, adj_new, adj_final)


# speedup vs baseline: 15.2097x; 1.0103x over previous
import functools

import jax
import jax.numpy as jnp
from jax.experimental import pallas as pl
from jax.experimental.pallas import tpu as pltpu

N = 10000
F = 256
H = 128
C = 40
TOPK = 50

_RB = 400
_RBD = 80

def _f2k(x):
    b = jax.lax.bitcast_convert_type(x, jnp.int32)
    imin = jnp.full_like(b, -2147483648)
    return jnp.where(b >= 0, b, imin - b)


def _k2f(k):
    imin = jnp.full_like(k, -2147483648)
    b = jnp.where(k >= 0, k, imin - k)
    return jax.lax.bitcast_convert_type(b, jnp.float32)


def _layer_kernel(a_ref, dr_ref, dc_ref, m_ref, o_ref):
    scaled = (dr_ref[...] * a_ref[...]) * dc_ref[...]
    o_ref[...] = jnp.dot(scaled, m_ref[...], preferred_element_type=jnp.float32)


def _layer(adj, dinv_col, dinv_row, m):
    return pl.pallas_call(
        _layer_kernel,
        grid=(N // _RB,),
        in_specs=[
            pl.BlockSpec((_RB, N), lambda i: (i, 0)),
            pl.BlockSpec((_RB, 1), lambda i: (i, 0)),
            pl.BlockSpec((1, N), lambda i: (0, 0)),
            pl.BlockSpec((N, F), lambda i: (0, 0)),
        ],
        out_specs=pl.BlockSpec((_RB, F), lambda i: (i, 0)),
        out_shape=jax.ShapeDtypeStruct((N, F), jnp.float32),
    )(adj, dinv_col, dinv_row, m)


def _simsel_kernel(e_ref, et_ref, tri_ref, t_ref, c_ref, sim_ref):
    e = e_ref[...]
    et = et_ref[...]
    half = F // 2
    sim = (jnp.dot(e[:, :half], et[:half, :], preferred_element_type=jnp.float32)
           + jnp.dot(e[:, half:], et[half:, :], preferred_element_type=jnp.float32))
    sim_ref[...] = sim

    rmax = jnp.max(sim, axis=1, keepdims=True)
    cmax = jnp.max(sim[:, :9984].reshape(_RB, 78, 128), axis=2)
    lo0 = _f2k(jnp.min(cmax, axis=1, keepdims=True))
    hi0 = _f2k(rmax) + 1

    def vcond(carry):
        lo, hi = carry
        return jnp.any(hi - lo > 1)

    def vbody(carry):
        lo, hi = carry
        mid = jax.lax.shift_right_arithmetic(lo + hi, 1)
        cnt = jnp.sum((sim_ref[...] >= _k2f(mid)).astype(jnp.float32),
                      axis=1, keepdims=True)
        feas = cnt >= TOPK
        return (jnp.where(feas, mid, lo), jnp.where(feas, hi, mid))

    lo, _ = jax.lax.while_loop(vcond, vbody, (lo0, hi0))
    t = _k2f(lo)

    simv = sim_ref[...]
    n_gt = jnp.sum((simv > t).astype(jnp.float32), axis=1, keepdims=True)
    n_tie = jnp.float32(TOPK) - n_gt

    colf = jax.lax.broadcasted_iota(jnp.int32, simv.shape, 1).astype(jnp.float32)
    eqf = (simv == t).astype(jnp.float32)
    val = jnp.where(simv == t, colf, jnp.float32(3e9))

    pref = jnp.dot(eqf, tri_ref[...], preferred_element_type=jnp.float32)
    kstar = jnp.sum((pref < n_tie).astype(jnp.int32), axis=1, keepdims=True)
    lo2 = kstar * 79 - 1
    hi2 = jnp.minimum(kstar * 79 + 78, N - 1)

    def ccond(carry):
        lo2, hi2 = carry
        return jnp.any(hi2 - lo2 > 1)

    def cbody(carry):
        lo2, hi2 = carry
        mid = jax.lax.shift_right_arithmetic(lo2 + hi2, 1)
        cnt = jnp.sum((val <= mid.astype(jnp.float32)).astype(jnp.float32),
                      axis=1, keepdims=True)
        feas = cnt >= n_tie
        return (jnp.where(feas, lo2, mid), jnp.where(feas, mid, hi2))

    _, hi2 = jax.lax.while_loop(ccond, cbody, (lo2, hi2))

    t_ref[...] = t
    c_ref[...] = hi2.astype(jnp.float32)


def _simsel(emb, embT, tri):
    return pl.pallas_call(
        _simsel_kernel,
        grid=(N // _RB,),
        in_specs=[
            pl.BlockSpec((_RB, F), lambda i: (i, 0)),
            pl.BlockSpec((F, N), lambda i: (0, 0)),
            pl.BlockSpec((N, 128), lambda i: (0, 0)),
        ],
        out_specs=[
            pl.BlockSpec((_RB, 1), lambda i: (i, 0)),
            pl.BlockSpec((_RB, 1), lambda i: (i, 0)),
        ],
        out_shape=[
            jax.ShapeDtypeStruct((N, 1), jnp.float32),
            jax.ShapeDtypeStruct((N, 1), jnp.float32),
        ],
        scratch_shapes=[pltpu.VMEM((_RB, N), jnp.float32)],
    )(emb, embT, tri)


def _graph_kernel(a_ref, e_ref, et_ref, tr_ref, cr_ref, tc_ref, cc_ref,
                  an_ref, af_ref, d2_ref):
    i = pl.program_id(0)
    e = e_ref[...]
    et = et_ref[...]
    half = F // 2
    sim = (jnp.dot(e[:, :half], et[:half, :], preferred_element_type=jnp.float32)
           + jnp.dot(e[:, half:], et[half:, :], preferred_element_type=jnp.float32))

    colf = jax.lax.broadcasted_iota(jnp.int32, sim.shape, 1).astype(jnp.float32)
    rowf = (jax.lax.broadcasted_iota(jnp.int32, sim.shape, 0).astype(jnp.float32)
            + jnp.float32(_RBD) * i.astype(jnp.float32))

    tr = tr_ref[...]
    cr = cr_ref[...]
    tc = tc_ref[...]
    cc = cc_ref[...]

    keep_r = jnp.logical_or(sim > tr, jnp.logical_and(sim == tr, colf <= cr))
    keep_c = jnp.logical_or(sim > tc, jnp.logical_and(sim == tc, rowf <= cc))
    fac = 0.5 * (keep_r.astype(jnp.float32) + keep_c.astype(jnp.float32))
    anew = sim * fac
    af = anew + a_ref[...]
    an_ref[...] = anew
    af_ref[...] = af
    d2_ref[...] = jnp.sum(af, axis=1, keepdims=True)


def _graph(adj, emb, embT, t_col, c_col, t_row, c_row):
    return pl.pallas_call(
        _graph_kernel,
        grid=(N // _RBD,),
        in_specs=[
            pl.BlockSpec((_RBD, N), lambda i: (i, 0)),
            pl.BlockSpec((_RBD, F), lambda i: (i, 0)),
            pl.BlockSpec((F, N), lambda i: (0, 0)),
            pl.BlockSpec((_RBD, 1), lambda i: (i, 0)),
            pl.BlockSpec((_RBD, 1), lambda i: (i, 0)),
            pl.BlockSpec((1, N), lambda i: (0, 0)),
            pl.BlockSpec((1, N), lambda i: (0, 0)),
        ],
        out_specs=[
            pl.BlockSpec((_RBD, N), lambda i: (i, 0)),
            pl.BlockSpec((_RBD, N), lambda i: (i, 0)),
            pl.BlockSpec((_RBD, 1), lambda i: (i, 0)),
        ],
        out_shape=[
            jax.ShapeDtypeStruct((N, N), jnp.float32),
            jax.ShapeDtypeStruct((N, N), jnp.float32),
            jax.ShapeDtypeStruct((N, 1), jnp.float32),
        ],
    )(adj, emb, embT, t_col, c_col, t_row, c_row)


def _mm_kernel(x_ref, w_ref, o_ref):
    o_ref[...] = jnp.dot(x_ref[...], w_ref[...],
                         preferred_element_type=jnp.float32)


def _mm(x, w):
    kin, nout = w.shape
    return pl.pallas_call(
        _mm_kernel,
        grid=(N // _RB,),
        in_specs=[
            pl.BlockSpec((_RB, kin), lambda i: (i, 0)),
            pl.BlockSpec((kin, nout), lambda i: (0, 0)),
        ],
        out_specs=pl.BlockSpec((_RB, nout), lambda i: (i, 0)),
        out_shape=jax.ShapeDtypeStruct((N, nout), jnp.float32),
    )(x, w)


def _enc_kernel(relu, af_ref, z_ref, d_ref, b_ref, o_ref):
    acc = jnp.dot(af_ref[...], z_ref[...], preferred_element_type=jnp.float32)
    r = d_ref[...] * acc + b_ref[...]
    if relu:
        r = jnp.maximum(r, 0.0)
    o_ref[...] = r


def _enc(af, z, dinv2, b, relu):
    nout = z.shape[1]
    return pl.pallas_call(
        functools.partial(_enc_kernel, relu),
        grid=(N // _RB,),
        in_specs=[
            pl.BlockSpec((_RB, N), lambda i: (i, 0)),
            pl.BlockSpec((N, nout), lambda i: (0, 0)),
            pl.BlockSpec((_RB, 1), lambda i: (i, 0)),
            pl.BlockSpec((1, nout), lambda i: (0, 0)),
        ],
        out_specs=pl.BlockSpec((_RB, nout), lambda i: (i, 0)),
        out_shape=jax.ShapeDtypeStruct((N, nout), jnp.float32),
    )(af, z, dinv2, b)


def kernel(input, Adj, w_diag1, w_diag2, W1, b1, W2, b2):
    deg = jnp.sum(Adj, axis=1)
    dinv = jnp.where(deg > 0, deg ** -0.5, 0.0)
    dcol = dinv.reshape(N, 1)
    drow = dinv.reshape(1, N)

    h = _layer(Adj, dcol, drow, input * w_diag1)
    h = _layer(Adj, dcol, drow, h * w_diag2)

    nrm = jnp.linalg.norm(h, axis=1, keepdims=True)
    emb = h / jnp.maximum(nrm, 1e-12)
    embT = emb.T

    tri = (jnp.arange(N, dtype=jnp.int32)[:, None]
           <= (79 * jnp.arange(128, dtype=jnp.int32) + 78)[None, :]
           ).astype(jnp.float32)
    t, cstar = _simsel(emb, embT, tri)

    adj_new, adj_final, deg2 = _graph(
        Adj, emb, embT, t, cstar, t.reshape(1, N), cstar.reshape(1, N))

    dinv2 = jnp.where(deg2 > 0, deg2 ** -0.5, 0.0)

    z1 = _mm(input, W1) * dinv2
    h1 = _enc(adj_final, z1, dinv2, b1.reshape(1, H), True)
    z2 = _mm(h1, W2) * dinv2
    out = _enc(adj_final, z2, dinv2, b2.reshape(1, C), False)

    return (out, adj_new, adj_final)
